# Initial kernel scaffold; baseline (speedup 1.0000x reference)
import jax
import jax.numpy as jnp
from jax.experimental import pallas as pl


def kernel(adjacency, nodes, edges, W_msg, W_upd, W_out):
    raise NotImplementedError("write your pallas kernel here")


# speedup vs baseline: 94.4444x; 94.4444x over previous
import jax
import jax.numpy as jnp
from jax.experimental import pallas as pl

B, N = 32, 32
NODE_F, EDGE_F, MSG, PASSES, OUT_F = 128, 16, 128, 3, 128
BN = B * N
NEF = N * EDGE_F


def _mpnn_kernel(a3_ref, aexp_ref, h_ref, er_ref, wmsg_ref, w3t_ref,
                 wupd_ref, wout_ref, out_ref):
    A3 = a3_ref[:]
    Aexp = aexp_ref[:]
    H0 = h_ref[:]
    Er = er_ref[:]
    W1 = wmsg_ref[0:NODE_F, :]
    W2 = wmsg_ref[NODE_F:2 * NODE_F, :]
    W3t = w3t_ref[:]
    Wu1 = wupd_ref[0:NODE_F, :]
    Wu2 = wupd_ref[NODE_F:, :]
    Wo1 = wout_ref[0:NODE_F, :]
    Wo2 = wout_ref[NODE_F:, :]

    deg = jnp.sum(Aexp, axis=1, keepdims=True) * (1.0 / EDGE_F)
    mask = (deg > 0.0).astype(jnp.float32)
    E3 = jnp.dot(Aexp * Er, W3t, preferred_element_type=jnp.float32)

    H = H0
    for _ in range(PASSES):
        Hb = H.reshape(B, N, NODE_F)
        neigh = jax.lax.dot_general(
            A3, Hb, (((2,), (1,)), ((0,), (0,))),
            preferred_element_type=jnp.float32).reshape(BN, NODE_F)
        msg = deg * jnp.dot(H, W1, preferred_element_type=jnp.float32) \
            + jnp.dot(neigh, W2, preferred_element_type=jnp.float32) + E3
        new = jnp.tanh(jnp.dot(H, Wu1, preferred_element_type=jnp.float32)
                       + jnp.dot(msg, Wu2, preferred_element_type=jnp.float32))
        H = mask * new + (1.0 - mask) * H

    G1 = jnp.sum((H * mask).reshape(B, N, NODE_F), axis=1)
    G2 = jnp.sum((H0 * mask).reshape(B, N, NODE_F), axis=1)
    out_ref[:] = (jnp.dot(G1, Wo1, preferred_element_type=jnp.float32)
                  + jnp.dot(G2, Wo2, preferred_element_type=jnp.float32))


def kernel(adjacency, nodes, edges, W_msg, W_upd, W_out):
    aexp = jnp.broadcast_to(adjacency.reshape(B, N, N, 1),
                            (B, N, N, EDGE_F)).reshape(BN, NEF)
    h = nodes.reshape(BN, NODE_F)
    er = edges.reshape(BN, NEF)
    w3t = jnp.tile(W_msg[2 * NODE_F:], (N, 1))
    return pl.pallas_call(
        _mpnn_kernel,
        out_shape=jax.ShapeDtypeStruct((B, OUT_F), jnp.float32),
    )(adjacency, aexp, h, er, W_msg, w3t, W_upd, W_out)


# speedup vs baseline: 147.8961x; 1.5660x over previous
import jax
import jax.numpy as jnp
from jax.experimental import pallas as pl

B, N = 32, 32
NODE_F, EDGE_F, MSG, PASSES, OUT_F = 128, 16, 128, 3, 128
BN = B * N
NEF = N * EDGE_F


def _mpnn_kernel(a_ref, h_ref, er_ref, wmsg_ref, wupd_ref, wout_ref, out_ref):
    Af = a_ref[:]
    H0 = h_ref[:]
    Er = er_ref[:]
    W1 = wmsg_ref[0:NODE_F, :]
    W2 = wmsg_ref[NODE_F:2 * NODE_F, :]
    W3 = wmsg_ref[2 * NODE_F:, :]
    Wu1 = wupd_ref[0:NODE_F, :]
    Wu2 = wupd_ref[NODE_F:, :]
    Wo1 = wout_ref[0:NODE_F, :]
    Wo2 = wout_ref[NODE_F:, :]

    f32 = jnp.float32
    r_row = jax.lax.broadcasted_iota(jnp.int32, (N, NEF), 0)
    r_col = jax.lax.broadcasted_iota(jnp.int32, (N, NEF), 1)
    R = (r_col // EDGE_F == r_row).astype(f32)
    t_row = jax.lax.broadcasted_iota(jnp.int32, (NEF, EDGE_F), 0)
    t_col = jax.lax.broadcasted_iota(jnp.int32, (NEF, EDGE_F), 1)
    Rt = (t_row % EDGE_F == t_col).astype(f32)

    Aexp = jnp.dot(Af, R, preferred_element_type=f32)
    EA = jnp.dot(Aexp * Er, Rt, preferred_element_type=f32)
    E3 = jnp.dot(EA, W3, preferred_element_type=f32)

    deg = jnp.sum(Af, axis=1, keepdims=True)
    maskb = deg > 0.0
    maskf = maskb.astype(f32)
    A3 = Af.reshape(B, N, N)

    H = H0
    for _ in range(PASSES):
        Hb = H.reshape(B, N, NODE_F)
        neigh = jax.lax.dot_general(
            A3, Hb, (((2,), (1,)), ((0,), (0,))),
            preferred_element_type=f32).reshape(BN, NODE_F)
        msg = deg * jnp.dot(H, W1, preferred_element_type=f32) \
            + jnp.dot(neigh, W2, preferred_element_type=f32) + E3
        new = jnp.tanh(jnp.dot(H, Wu1, preferred_element_type=f32)
                       + jnp.dot(msg, Wu2, preferred_element_type=f32))
        H = jnp.where(maskb, new, H)

    G1 = jnp.sum((H * maskf).reshape(B, N, NODE_F), axis=1)
    G2 = jnp.sum((H0 * maskf).reshape(B, N, NODE_F), axis=1)
    out_ref[:] = (jnp.dot(G1, Wo1, preferred_element_type=f32)
                  + jnp.dot(G2, Wo2, preferred_element_type=f32))


def kernel(adjacency, nodes, edges, W_msg, W_upd, W_out):
    return pl.pallas_call(
        _mpnn_kernel,
        out_shape=jax.ShapeDtypeStruct((B, OUT_F), jnp.float32),
    )(adjacency.reshape(BN, N), nodes.reshape(BN, NODE_F),
      edges.reshape(BN, NEF), W_msg, W_upd, W_out)
